# Initial kernel scaffold; baseline (speedup 1.0000x reference)
import jax
import jax.numpy as jnp
from jax.experimental import pallas as pl


def kernel(x, neighbor_index, neighbor_weight, neighbor_field, W_poly):
    raise NotImplementedError("write your pallas kernel here")


# speedup vs baseline: 3.8932x; 3.8932x over previous
import functools

import jax
import jax.numpy as jnp
from jax import lax
from jax.experimental import pallas as pl
from jax.experimental.pallas import tpu as pltpu
from jax.experimental.pallas import tpu_sc as plsc

N = 10000
DEG = 16
C = 128
K = 3
G = 8
VCH = C // 16
NCHUNKS = N // G
NW = 32


def _sc_body(x_hbm, idx_hbm, field_hbm, w_hbm, out_hbm,
             idxv, rows, wv, fsm, outv, gsem):
    nc = 2
    wid = lax.axis_index("s") * nc + lax.axis_index("c")
    nch = (NCHUNKS - wid + NW - 1) // NW
    pltpu.sync_copy(w_hbm, wv)

    def chunk(i, _):
        cb = wid + i * NW
        pltpu.sync_copy(idx_hbm.at[pl.ds(cb * G * DEG, G * DEG)], idxv)
        pltpu.sync_copy(field_hbm.at[pl.ds(cb * G, 16)], fsm)
        pltpu.async_copy(x_hbm.at[idxv], rows, gsem).wait()

        def node(g, _):
            def dstep(d, acc):
                r = g * DEG + d
                s1 = []
                s2 = []
                for v in range(VCH):
                    vec = rows[r, pl.ds(v * 16, 16)]
                    s1.append(acc[v] + vec)
                    s2.append(acc[VCH + v] + vec * vec)
                return tuple(s1 + s2)

            zero = jnp.zeros((16,), jnp.float32)
            acc = lax.fori_loop(0, DEG, dstep, tuple([zero] * (2 * VCH)))
            fvec = fsm[...]
            dnums = lax.GatherDimensionNumbers(
                offset_dims=(), collapsed_slice_dims=(0,), start_index_map=(0,))
            f = lax.gather(fvec, jnp.full((16, 1), g, dtype=jnp.int32),
                           dnums, slice_sizes=(1,),
                           mode=lax.GatherScatterMode.PROMISE_IN_BOUNDS)
            ob = g * (K * C)
            for v in range(VCH):
                sl = pl.ds(v * 16, 16)
                w0 = wv[0, sl]
                w1 = wv[1, sl]
                w2a = wv[2, sl]
                w2b = wv[3, sl]
                outv[pl.ds(ob + v * 16, 16)] = f * w0
                outv[pl.ds(ob + C + v * 16, 16)] = (f * acc[v]) * w1
                outv[pl.ds(ob + 2 * C + v * 16, 16)] = f * (acc[VCH + v] * w2a - w2b)
            return 0

        lax.fori_loop(0, G, node, 0)
        pltpu.sync_copy(outv, out_hbm.at[pl.ds(cb * G * K * C, G * K * C)])
        return 0

    lax.fori_loop(0, nch, chunk, 0)


def kernel(x, neighbor_index, neighbor_weight, neighbor_field, W_poly):
    del neighbor_weight
    x2 = x.reshape(N, C)
    idx_flat = neighbor_index.reshape(N * DEG).astype(jnp.int32)
    field = jnp.pad(neighbor_field.astype(jnp.float32), (0, 16))
    w0 = W_poly[:, 0]
    w1 = W_poly[:, 1]
    w2 = W_poly[:, 2]
    wmat = jnp.stack([16.0 * w0, 2.0 * w1, 4.0 * w2, 16.0 * w2])

    mesh = plsc.VectorSubcoreMesh(core_axis_name="c", subcore_axis_name="s")
    kfn = pl.kernel(
        _sc_body,
        out_type=jax.ShapeDtypeStruct((N * K * C,), jnp.float32),
        mesh=mesh,
        scratch_types=[
            pltpu.VMEM((G * DEG,), jnp.int32),
            pltpu.VMEM((G * DEG, C), jnp.float32),
            pltpu.VMEM((4, C), jnp.float32),
            pltpu.VMEM((16,), jnp.float32),
            pltpu.VMEM((G * K * C,), jnp.float32),
            pltpu.SemaphoreType.DMA,
        ],
    )
    out_flat = kfn(x2, idx_flat, field, wmat)
    return out_flat.reshape(1, N, K, C)


# speedup vs baseline: 4.7918x; 1.2308x over previous
import jax
import jax.numpy as jnp
from jax import lax
from jax.experimental import pallas as pl
from jax.experimental.pallas import tpu as pltpu
from jax.experimental.pallas import tpu_sc as plsc

N = 10000
DEG = 16
C = 128
K = 3
G = 8
GI = G * DEG
GKC = G * K * C
VCH = C // 16
NCHUNKS = N // G
NW = 32
MAXCW = 40
NPAIRS = MAXCW // 2


def _sc_body(x_hbm, idx_hbm, field_hbm, w_hbm, out_hbm,
             idxa, fsa, wv, rows0, rows1, outv0, outv1,
             gsem0, gsem1, osem0, osem1):
    nc = 2
    wid = lax.axis_index("s") * nc + lax.axis_index("c")
    ncw = jnp.where(wid < 2, MAXCW, MAXCW - 1)
    cbase = wid * (MAXCW - 1) + jnp.minimum(wid, 2)

    pltpu.sync_copy(w_hbm, wv)
    pltpu.sync_copy(idx_hbm.at[pl.ds(cbase * GI, MAXCW * GI)], idxa)
    pltpu.sync_copy(field_hbm.at[pl.ds(cbase * G, MAXCW * G + 16)], fsa)

    def fire_gather(j, rows, gsem):
        pltpu.async_copy(x_hbm.at[idxa.at[pl.ds(j * GI, GI)]], rows, gsem)

    def wait_gather(rows, gsem):
        pltpu.make_async_copy(x_hbm.at[idxa.at[pl.ds(0, GI)]], rows, gsem).wait()

    def wait_out(outv, osem):
        pltpu.make_async_copy(outv, out_hbm.at[pl.ds(0, GKC)], osem).wait()

    def compute(j, rows, outv):
        fvec = fsa[pl.ds(j * G, 16)]

        def node(g, _):
            rb = g * DEG
            acc1 = [jnp.zeros((16,), jnp.float32)] * VCH
            acc2 = [jnp.zeros((16,), jnp.float32)] * VCH
            for d in range(DEG):
                for v in range(VCH):
                    vec = rows[rb + d, pl.ds(v * 16, 16)]
                    acc1[v] = acc1[v] + vec
                    acc2[v] = acc2[v] + vec * vec
            dnums = lax.GatherDimensionNumbers(
                offset_dims=(), collapsed_slice_dims=(0,), start_index_map=(0,))
            f = lax.gather(fvec, jnp.full((16, 1), g, dtype=jnp.int32),
                           dnums, slice_sizes=(1,),
                           mode=lax.GatherScatterMode.PROMISE_IN_BOUNDS)
            ob = g * (K * C)
            for v in range(VCH):
                sl = pl.ds(v * 16, 16)
                outv[pl.ds(ob + v * 16, 16)] = f * wv[0, sl]
                outv[pl.ds(ob + C + v * 16, 16)] = (f * acc1[v]) * wv[1, sl]
                outv[pl.ds(ob + 2 * C + v * 16, 16)] = \
                    f * (acc2[v] * wv[2, sl] - wv[3, sl])
            return 0

        lax.fori_loop(0, G, node, 0)

    def fire_out(j, outv, osem):
        pltpu.async_copy(outv, out_hbm.at[pl.ds((cbase + j) * GKC, GKC)], osem)

    fire_gather(0, rows0, gsem0)

    def pair(p, _):
        j0 = 2 * p
        j1 = j0 + 1

        @pl.when(j1 < ncw)
        def _():
            fire_gather(j1, rows1, gsem1)

        wait_gather(rows0, gsem0)

        @pl.when(p > 0)
        def _():
            wait_out(outv0, osem0)

        compute(j0, rows0, outv0)
        fire_out(j0, outv0, osem0)

        @pl.when(j0 + 2 < ncw)
        def _():
            fire_gather(j0 + 2, rows0, gsem0)

        @pl.when(j1 < ncw)
        def _():
            wait_gather(rows1, gsem1)

            @pl.when(p > 0)
            def _():
                wait_out(outv1, osem1)

            compute(j1, rows1, outv1)
            fire_out(j1, outv1, osem1)

        return 0

    lax.fori_loop(0, NPAIRS, pair, 0)
    wait_out(outv0, osem0)
    wait_out(outv1, osem1)


def kernel(x, neighbor_index, neighbor_weight, neighbor_field, W_poly):
    del neighbor_weight
    x2 = x.reshape(N, C)
    idx_flat = jnp.pad(neighbor_index.reshape(N * DEG).astype(jnp.int32),
                       (0, MAXCW * GI))
    field = jnp.pad(neighbor_field.astype(jnp.float32), (0, MAXCW * G + 16))
    w0 = W_poly[:, 0]
    w1 = W_poly[:, 1]
    w2 = W_poly[:, 2]
    wmat = jnp.stack([16.0 * w0, 2.0 * w1, 4.0 * w2, 16.0 * w2])

    mesh = plsc.VectorSubcoreMesh(core_axis_name="c", subcore_axis_name="s")
    kfn = pl.kernel(
        _sc_body,
        out_type=jax.ShapeDtypeStruct((N * K * C,), jnp.float32),
        mesh=mesh,
        scratch_types=[
            pltpu.VMEM((MAXCW * GI,), jnp.int32),
            pltpu.VMEM((MAXCW * G + 16,), jnp.float32),
            pltpu.VMEM((4, C), jnp.float32),
            pltpu.VMEM((GI, C), jnp.float32),
            pltpu.VMEM((GI, C), jnp.float32),
            pltpu.VMEM((GKC,), jnp.float32),
            pltpu.VMEM((GKC,), jnp.float32),
            pltpu.SemaphoreType.DMA,
            pltpu.SemaphoreType.DMA,
            pltpu.SemaphoreType.DMA,
            pltpu.SemaphoreType.DMA,
        ],
    )
    out_flat = kfn(x2, idx_flat, field, wmat)
    return out_flat.reshape(1, N, K, C)


# speedup vs baseline: 6.6291x; 1.3834x over previous
import jax
import jax.numpy as jnp
from jax import lax
from jax.experimental import pallas as pl
from jax.experimental.pallas import tpu as pltpu
from jax.experimental.pallas import tpu_sc as plsc

N = 10000
DEG = 16
C = 128
K = 3
G = 8
GI = G * DEG
GKC = G * K * C
VCH = C // 16
NCHUNKS = N // G
NW = 32
MAXCW = 40
NPAIRS = MAXCW // 2


def _sc_body(x_hbm, idx_hbm, field_hbm, w_hbm, out_hbm,
             idxa, fsa, wv, rows0, rows1, outv0, outv1,
             gsem0, gsem1, osem0, osem1):
    nc = 2
    wid = lax.axis_index("s") * nc + lax.axis_index("c")
    ncw = jnp.where(wid < 2, MAXCW, MAXCW - 1)
    cbase = wid * (MAXCW - 1) + jnp.minimum(wid, 2)

    pltpu.sync_copy(w_hbm, wv)
    pltpu.sync_copy(idx_hbm.at[pl.ds(cbase * GI, MAXCW * GI)], idxa)
    pltpu.sync_copy(field_hbm.at[pl.ds(cbase * G, MAXCW * G + 16)], fsa)

    def fire_gather(j, rows, gsem):
        pltpu.async_copy(x_hbm.at[idxa.at[pl.ds(j * GI, GI)]], rows, gsem)

    def wait_gather(rows, gsem):
        pltpu.make_async_copy(x_hbm.at[idxa.at[pl.ds(0, GI)]], rows, gsem).wait()

    def wait_out(outv, osem):
        pltpu.make_async_copy(outv, out_hbm.at[pl.ds(0, GKC)], osem).wait()

    def compute(j, rows, outv):
        fvec = fsa[pl.ds(j * G, 16)]
        wvec = [[wv[r, pl.ds(v * 16, 16)] for v in range(VCH)] for r in range(4)]

        def node(g, _):
            rb = g * DEG
            dnums = lax.GatherDimensionNumbers(
                offset_dims=(), collapsed_slice_dims=(0,), start_index_map=(0,))
            f = lax.gather(fvec, jnp.full((16, 1), g, dtype=jnp.int32),
                           dnums, slice_sizes=(1,),
                           mode=lax.GatherScatterMode.PROMISE_IN_BOUNDS)
            ob = g * (K * C)
            for v in range(VCH):
                sl = pl.ds(v * 16, 16)
                p1 = [rows[rb + d, sl] for d in range(4)]
                p2 = [p * p for p in p1]
                for d in range(4, DEG):
                    vec = rows[rb + d, sl]
                    q = d & 3
                    p1[q] = p1[q] + vec
                    p2[q] = p2[q] + vec * vec
                s1 = (p1[0] + p1[1]) + (p1[2] + p1[3])
                s2 = (p2[0] + p2[1]) + (p2[2] + p2[3])
                outv[pl.ds(ob + v * 16, 16)] = f * wvec[0][v]
                outv[pl.ds(ob + C + v * 16, 16)] = (f * s1) * wvec[1][v]
                outv[pl.ds(ob + 2 * C + v * 16, 16)] = \
                    f * (s2 * wvec[2][v] - wvec[3][v])
            return 0

        lax.fori_loop(0, G, node, 0)

    def fire_out(j, outv, osem):
        pltpu.async_copy(outv, out_hbm.at[pl.ds((cbase + j) * GKC, GKC)], osem)

    fire_gather(0, rows0, gsem0)

    def pair(p, _):
        j0 = 2 * p
        j1 = j0 + 1

        @pl.when(j1 < ncw)
        def _():
            fire_gather(j1, rows1, gsem1)

        wait_gather(rows0, gsem0)

        @pl.when(p > 0)
        def _():
            wait_out(outv0, osem0)

        compute(j0, rows0, outv0)
        fire_out(j0, outv0, osem0)

        @pl.when(j0 + 2 < ncw)
        def _():
            fire_gather(j0 + 2, rows0, gsem0)

        @pl.when(j1 < ncw)
        def _():
            wait_gather(rows1, gsem1)

            @pl.when(p > 0)
            def _():
                wait_out(outv1, osem1)

            compute(j1, rows1, outv1)
            fire_out(j1, outv1, osem1)

        return 0

    lax.fori_loop(0, NPAIRS, pair, 0)
    wait_out(outv0, osem0)
    wait_out(outv1, osem1)


def kernel(x, neighbor_index, neighbor_weight, neighbor_field, W_poly):
    del neighbor_weight
    x2 = x.reshape(N, C)
    idx_flat = jnp.pad(neighbor_index.reshape(N * DEG).astype(jnp.int32),
                       (0, MAXCW * GI))
    field = jnp.pad(neighbor_field.astype(jnp.float32), (0, MAXCW * G + 16))
    w0 = W_poly[:, 0]
    w1 = W_poly[:, 1]
    w2 = W_poly[:, 2]
    wmat = jnp.stack([16.0 * w0, 2.0 * w1, 4.0 * w2, 16.0 * w2])

    mesh = plsc.VectorSubcoreMesh(core_axis_name="c", subcore_axis_name="s")
    kfn = pl.kernel(
        _sc_body,
        out_type=jax.ShapeDtypeStruct((N * K * C,), jnp.float32),
        mesh=mesh,
        scratch_types=[
            pltpu.VMEM((MAXCW * GI,), jnp.int32),
            pltpu.VMEM((MAXCW * G + 16,), jnp.float32),
            pltpu.VMEM((4, C), jnp.float32),
            pltpu.VMEM((GI, C), jnp.float32),
            pltpu.VMEM((GI, C), jnp.float32),
            pltpu.VMEM((GKC,), jnp.float32),
            pltpu.VMEM((GKC,), jnp.float32),
            pltpu.SemaphoreType.DMA,
            pltpu.SemaphoreType.DMA,
            pltpu.SemaphoreType.DMA,
            pltpu.SemaphoreType.DMA,
        ],
    )
    out_flat = kfn(x2, idx_flat, field, wmat)
    return out_flat.reshape(1, N, K, C)


# speedup vs baseline: 7.3149x; 1.1035x over previous
import jax
import jax.numpy as jnp
from jax import lax
from jax.experimental import pallas as pl
from jax.experimental.pallas import tpu as pltpu
from jax.experimental.pallas import tpu_sc as plsc

N = 10000
DEG = 16
C = 128
K = 3
G = 8
GI = G * DEG
GKC = G * K * C
VCH = C // 16
NCHUNKS = N // G
NW = 32
MAXCW = 40
NPAIRS = MAXCW // 2


def _sc_body(x_hbm, idx_hbm, field_hbm, w_hbm, out_hbm,
             idxa, fsa, wv, rows0, rows1, outv0, outv1,
             gsem0, gsem1, osem0, osem1):
    nc = 2
    wid = lax.axis_index("s") * nc + lax.axis_index("c")
    ncw = jnp.where(wid < 2, MAXCW, MAXCW - 1)
    cbase = wid * (MAXCW - 1) + jnp.minimum(wid, 2)

    pltpu.sync_copy(w_hbm, wv)
    pltpu.sync_copy(idx_hbm.at[pl.ds(cbase * GI, MAXCW * GI)], idxa)
    pltpu.sync_copy(field_hbm.at[pl.ds(cbase * G, MAXCW * G + 16)], fsa)

    def fire_gather(j, rows, gsem):
        pltpu.async_copy(x_hbm.at[idxa.at[pl.ds(j * GI, GI)]], rows, gsem)

    def wait_gather(rows, gsem):
        pltpu.make_async_copy(x_hbm.at[idxa.at[pl.ds(0, GI)]], rows, gsem).wait()

    def wait_out(outv, osem):
        pltpu.make_async_copy(outv, out_hbm.at[pl.ds(0, GKC)], osem).wait()

    def compute(j, rows, outv):
        fvec = fsa[pl.ds(j * G, 16)]
        wvec = [[wv[r, pl.ds(v * 16, 16)] for v in range(VCH)] for r in range(4)]

        def node(g, _):
            rb = g * DEG
            dnums = lax.GatherDimensionNumbers(
                offset_dims=(), collapsed_slice_dims=(0,), start_index_map=(0,))
            f = lax.gather(fvec, jnp.full((16, 1), g, dtype=jnp.int32),
                           dnums, slice_sizes=(1,),
                           mode=lax.GatherScatterMode.PROMISE_IN_BOUNDS)
            ob = g * (K * C)
            outs = []
            for v in range(VCH):
                sl = pl.ds(v * 16, 16)
                p1 = [rows[rb + d, sl] for d in range(4)]
                p2 = [p * p for p in p1]
                for d in range(4, DEG):
                    vec = rows[rb + d, sl]
                    q = d & 3
                    p1[q] = p1[q] + vec
                    p2[q] = p2[q] + vec * vec
                s1 = (p1[0] + p1[1]) + (p1[2] + p1[3])
                s2 = (p2[0] + p2[1]) + (p2[2] + p2[3])
                outs.append((v, f * wvec[0][v], (f * s1) * wvec[1][v],
                             f * (s2 * wvec[2][v] - wvec[3][v])))
            for v, o0, o1, o2 in outs:
                outv[pl.ds(ob + v * 16, 16)] = o0
                outv[pl.ds(ob + C + v * 16, 16)] = o1
                outv[pl.ds(ob + 2 * C + v * 16, 16)] = o2
            return 0

        lax.fori_loop(0, G, node, 0)

    def fire_out(j, outv, osem):
        pltpu.async_copy(outv, out_hbm.at[pl.ds((cbase + j) * GKC, GKC)], osem)

    fire_gather(0, rows0, gsem0)

    def pair(p, _):
        j0 = 2 * p
        j1 = j0 + 1

        @pl.when(j1 < ncw)
        def _():
            fire_gather(j1, rows1, gsem1)

        wait_gather(rows0, gsem0)

        @pl.when(p > 0)
        def _():
            wait_out(outv0, osem0)

        compute(j0, rows0, outv0)
        fire_out(j0, outv0, osem0)

        @pl.when(j0 + 2 < ncw)
        def _():
            fire_gather(j0 + 2, rows0, gsem0)

        @pl.when(j1 < ncw)
        def _():
            wait_gather(rows1, gsem1)

            @pl.when(p > 0)
            def _():
                wait_out(outv1, osem1)

            compute(j1, rows1, outv1)
            fire_out(j1, outv1, osem1)

        return 0

    lax.fori_loop(0, NPAIRS, pair, 0)
    wait_out(outv0, osem0)
    wait_out(outv1, osem1)


def kernel(x, neighbor_index, neighbor_weight, neighbor_field, W_poly):
    del neighbor_weight
    x2 = x.reshape(N, C)
    idx_flat = jnp.pad(neighbor_index.reshape(N * DEG).astype(jnp.int32),
                       (0, MAXCW * GI))
    field = jnp.pad(neighbor_field.astype(jnp.float32), (0, MAXCW * G + 16))
    w0 = W_poly[:, 0]
    w1 = W_poly[:, 1]
    w2 = W_poly[:, 2]
    wmat = jnp.stack([16.0 * w0, 2.0 * w1, 4.0 * w2, 16.0 * w2])

    mesh = plsc.VectorSubcoreMesh(core_axis_name="c", subcore_axis_name="s")
    kfn = pl.kernel(
        _sc_body,
        out_type=jax.ShapeDtypeStruct((N * K * C,), jnp.float32),
        mesh=mesh,
        scratch_types=[
            pltpu.VMEM((MAXCW * GI,), jnp.int32),
            pltpu.VMEM((MAXCW * G + 16,), jnp.float32),
            pltpu.VMEM((4, C), jnp.float32),
            pltpu.VMEM((GI, C), jnp.float32),
            pltpu.VMEM((GI, C), jnp.float32),
            pltpu.VMEM((GKC,), jnp.float32),
            pltpu.VMEM((GKC,), jnp.float32),
            pltpu.SemaphoreType.DMA,
            pltpu.SemaphoreType.DMA,
            pltpu.SemaphoreType.DMA,
            pltpu.SemaphoreType.DMA,
        ],
    )
    out_flat = kfn(x2, idx_flat, field, wmat)
    return out_flat.reshape(1, N, K, C)


# speedup vs baseline: 7.3429x; 1.0038x over previous
import jax
import jax.numpy as jnp
from jax import lax
from jax.experimental import pallas as pl
from jax.experimental.pallas import tpu as pltpu
from jax.experimental.pallas import tpu_sc as plsc

N = 10000
DEG = 16
C = 128
K = 3
G = 8
GI = G * DEG
KP = 8
GKC = G * KP * C
VCH = C // 16
NCHUNKS = N // G
NW = 32
MAXCW = 40
NPAIRS = MAXCW // 2


def _sc_body(x_hbm, idx_hbm, field_hbm, w_hbm, out_hbm,
             idxa, fsa, wv, rows0, rows1, outv0, outv1,
             gsem0, gsem1, osem0, osem1):
    nc = 2
    wid = lax.axis_index("s") * nc + lax.axis_index("c")
    ncw = jnp.where(wid < 2, MAXCW, MAXCW - 1)
    cbase = wid * (MAXCW - 1) + jnp.minimum(wid, 2)

    pltpu.sync_copy(w_hbm, wv)
    pltpu.sync_copy(idx_hbm.at[pl.ds(cbase * GI, MAXCW * GI)], idxa)
    pltpu.sync_copy(field_hbm.at[pl.ds(cbase * G, MAXCW * G + 16)], fsa)

    def fire_gather(j, rows, gsem):
        pltpu.async_copy(x_hbm.at[idxa.at[pl.ds(j * GI, GI)]], rows, gsem)

    def wait_gather(rows, gsem):
        pltpu.make_async_copy(x_hbm.at[idxa.at[pl.ds(0, GI)]], rows, gsem).wait()

    def wait_out(outv, osem):
        pltpu.make_async_copy(outv, out_hbm.at[pl.ds(0, GKC)], osem).wait()

    def compute(j, rows, outv):
        fvec = fsa[pl.ds(j * G, 16)]
        wvec = [[wv[r, pl.ds(v * 16, 16)] for v in range(VCH)] for r in range(4)]

        def node(g, _):
            rb = g * DEG
            dnums = lax.GatherDimensionNumbers(
                offset_dims=(), collapsed_slice_dims=(0,), start_index_map=(0,))
            f = lax.gather(fvec, jnp.full((16, 1), g, dtype=jnp.int32),
                           dnums, slice_sizes=(1,),
                           mode=lax.GatherScatterMode.PROMISE_IN_BOUNDS)
            ob = g * (KP * C)
            outs = []
            for v in range(VCH):
                sl = pl.ds(v * 16, 16)
                p1 = [rows[rb + d, sl] for d in range(4)]
                p2 = [p * p for p in p1]
                for d in range(4, DEG):
                    vec = rows[rb + d, sl]
                    q = d & 3
                    p1[q] = p1[q] + vec
                    p2[q] = p2[q] + vec * vec
                s1 = (p1[0] + p1[1]) + (p1[2] + p1[3])
                s2 = (p2[0] + p2[1]) + (p2[2] + p2[3])
                outs.append((v, f * wvec[0][v], (f * s1) * wvec[1][v],
                             f * (s2 * wvec[2][v] - wvec[3][v])))
            for v, o0, o1, o2 in outs:
                outv[pl.ds(ob + v * 16, 16)] = o0
                outv[pl.ds(ob + C + v * 16, 16)] = o1
                outv[pl.ds(ob + 2 * C + v * 16, 16)] = o2
            return 0

        lax.fori_loop(0, G, node, 0)

    def fire_out(j, outv, osem):
        pltpu.async_copy(outv, out_hbm.at[pl.ds((cbase + j) * GKC, GKC)], osem)

    fire_gather(0, rows0, gsem0)

    def pair(p, _):
        j0 = 2 * p
        j1 = j0 + 1

        @pl.when(j1 < ncw)
        def _():
            fire_gather(j1, rows1, gsem1)

        wait_gather(rows0, gsem0)

        @pl.when(p > 0)
        def _():
            wait_out(outv0, osem0)

        compute(j0, rows0, outv0)
        fire_out(j0, outv0, osem0)

        @pl.when(j0 + 2 < ncw)
        def _():
            fire_gather(j0 + 2, rows0, gsem0)

        @pl.when(j1 < ncw)
        def _():
            wait_gather(rows1, gsem1)

            @pl.when(p > 0)
            def _():
                wait_out(outv1, osem1)

            compute(j1, rows1, outv1)
            fire_out(j1, outv1, osem1)

        return 0

    lax.fori_loop(0, NPAIRS, pair, 0)
    wait_out(outv0, osem0)
    wait_out(outv1, osem1)


def kernel(x, neighbor_index, neighbor_weight, neighbor_field, W_poly):
    del neighbor_weight
    x2 = x.reshape(N, C)
    idx_flat = jnp.pad(neighbor_index.reshape(N * DEG).astype(jnp.int32),
                       (0, MAXCW * GI))
    field = jnp.pad(neighbor_field.astype(jnp.float32), (0, MAXCW * G + 16))
    w0 = W_poly[:, 0]
    w1 = W_poly[:, 1]
    w2 = W_poly[:, 2]
    wmat = jnp.stack([16.0 * w0, 2.0 * w1, 4.0 * w2, 16.0 * w2])

    mesh = plsc.VectorSubcoreMesh(core_axis_name="c", subcore_axis_name="s")
    kfn = pl.kernel(
        _sc_body,
        out_type=jax.ShapeDtypeStruct((N * KP * C,), jnp.float32),
        mesh=mesh,
        scratch_types=[
            pltpu.VMEM((MAXCW * GI,), jnp.int32),
            pltpu.VMEM((MAXCW * G + 16,), jnp.float32),
            pltpu.VMEM((4, C), jnp.float32),
            pltpu.VMEM((GI, C), jnp.float32),
            pltpu.VMEM((GI, C), jnp.float32),
            pltpu.VMEM((GKC,), jnp.float32),
            pltpu.VMEM((GKC,), jnp.float32),
            pltpu.SemaphoreType.DMA,
            pltpu.SemaphoreType.DMA,
            pltpu.SemaphoreType.DMA,
            pltpu.SemaphoreType.DMA,
        ],
    )
    out_flat = kfn(x2, idx_flat, field, wmat)
    return out_flat.reshape(1, N, KP, C)[:, :, :K, :]


# speedup vs baseline: 7.9904x; 1.0882x over previous
import jax
import jax.numpy as jnp
from jax import lax
from jax.experimental import pallas as pl
from jax.experimental.pallas import tpu as pltpu
from jax.experimental.pallas import tpu_sc as plsc

N = 10000
DEG = 16
C = 128
K = 3
G = 8
GI = G * DEG
GKC = G * K * C
VCH = C // 16
NCHUNKS = N // G
NW = 32
MAXCW = 40
NQUADS = MAXCW // 4
NBUF = 4


def _sc_body(x_hbm, idx_hbm, field_hbm, w_hbm, out_hbm,
             idxa, fsa, wv, rows, outs, gsems, osems):
    nc = 2
    wid = lax.axis_index("s") * nc + lax.axis_index("c")
    ncw = jnp.where(wid < 2, MAXCW, MAXCW - 1)
    cbase = wid * (MAXCW - 1) + jnp.minimum(wid, 2)

    pltpu.sync_copy(w_hbm, wv)
    pltpu.sync_copy(idx_hbm.at[pl.ds(cbase * GI, MAXCW * GI)], idxa)
    pltpu.sync_copy(field_hbm.at[pl.ds(cbase * G, MAXCW * G + 16)], fsa)

    def fire_gather(j, b):
        pltpu.async_copy(x_hbm.at[idxa.at[pl.ds(j * GI, GI)]], rows[b], gsems[b])

    def wait_gather(b):
        pltpu.make_async_copy(x_hbm.at[idxa.at[pl.ds(0, GI)]], rows[b],
                              gsems[b]).wait()

    def wait_out(b):
        pltpu.make_async_copy(outs[b], out_hbm.at[pl.ds(0, GKC)], osems[b]).wait()

    def fire_out(j, b):
        pltpu.async_copy(outs[b], out_hbm.at[pl.ds((cbase + j) * GKC, GKC)],
                         osems[b])

    def compute(j, b):
        fvec = fsa[pl.ds(j * G, 16)]
        wvec = [[wv[r, pl.ds(v * 16, 16)] for v in range(VCH)] for r in range(4)]
        rbuf = rows[b]
        outv = outs[b]

        def node(g, _):
            rb = g * DEG
            dnums = lax.GatherDimensionNumbers(
                offset_dims=(), collapsed_slice_dims=(0,), start_index_map=(0,))
            f = lax.gather(fvec, jnp.full((16, 1), g, dtype=jnp.int32),
                           dnums, slice_sizes=(1,),
                           mode=lax.GatherScatterMode.PROMISE_IN_BOUNDS)
            ob = g * (K * C)
            res = []
            for v in range(VCH):
                sl = pl.ds(v * 16, 16)
                p1 = [rbuf[rb + d, sl] for d in range(4)]
                p2 = [p * p for p in p1]
                for d in range(4, DEG):
                    vec = rbuf[rb + d, sl]
                    q = d & 3
                    p1[q] = p1[q] + vec
                    p2[q] = p2[q] + vec * vec
                s1 = (p1[0] + p1[1]) + (p1[2] + p1[3])
                s2 = (p2[0] + p2[1]) + (p2[2] + p2[3])
                res.append((v, f * wvec[0][v], (f * s1) * wvec[1][v],
                            f * (s2 * wvec[2][v] - wvec[3][v])))
            for v, o0, o1, o2 in res:
                outv[pl.ds(ob + v * 16, 16)] = o0
                outv[pl.ds(ob + C + v * 16, 16)] = o1
                outv[pl.ds(ob + 2 * C + v * 16, 16)] = o2
            return 0

        lax.fori_loop(0, G, node, 0)

    fire_gather(0, 0)
    fire_gather(1, 1)
    fire_gather(2, 2)

    def quad(p, _):
        for q in range(4):
            j = 4 * p + q

            def step():
                wait_gather(q)

                @pl.when(p > 0)
                def _():
                    wait_out(q)

                compute(j, q)
                fire_out(j, q)

                @pl.when(j + 3 < ncw)
                def _():
                    fire_gather(j + 3, (q + 3) % NBUF)

            if q < 3:
                step()
            else:
                pl.when(j < ncw)(step)
        return 0

    lax.fori_loop(0, NQUADS, quad, 0)
    wait_out(0)
    wait_out(1)
    wait_out(2)

    @pl.when(ncw == MAXCW)
    def _():
        wait_out(3)


def kernel(x, neighbor_index, neighbor_weight, neighbor_field, W_poly):
    del neighbor_weight
    x2 = x.reshape(N, C)
    idx_flat = jnp.pad(neighbor_index.reshape(N * DEG).astype(jnp.int32),
                       (0, MAXCW * GI))
    field = jnp.pad(neighbor_field.astype(jnp.float32), (0, MAXCW * G + 16))
    w0 = W_poly[:, 0]
    w1 = W_poly[:, 1]
    w2 = W_poly[:, 2]
    wmat = jnp.stack([16.0 * w0, 2.0 * w1, 4.0 * w2, 16.0 * w2])

    mesh = plsc.VectorSubcoreMesh(core_axis_name="c", subcore_axis_name="s")
    kfn = pl.kernel(
        _sc_body,
        out_type=jax.ShapeDtypeStruct((N * K * C,), jnp.float32),
        mesh=mesh,
        scratch_types=[
            pltpu.VMEM((MAXCW * GI,), jnp.int32),
            pltpu.VMEM((MAXCW * G + 16,), jnp.float32),
            pltpu.VMEM((4, C), jnp.float32),
            [pltpu.VMEM((GI, C), jnp.float32) for _ in range(NBUF)],
            [pltpu.VMEM((GKC,), jnp.float32) for _ in range(NBUF)],
            [pltpu.SemaphoreType.DMA for _ in range(NBUF)],
            [pltpu.SemaphoreType.DMA for _ in range(NBUF)],
        ],
    )
    out_flat = kfn(x2, idx_flat, field, wmat)
    return out_flat.reshape(1, N, K, C)
